# baseline (speedup 1.0000x reference)
import functools

import jax
import jax.numpy as jnp
import numpy as np
from jax import lax
from jax.experimental import pallas as pl
from jax.experimental.pallas import tpu as pltpu
from jax.experimental.pallas import tpu_sc as plsc

_DIMS = np.array(
    [6, 7, 8, 9, 10, 11, 12, 13, 14, 15, 16, 17, 21, 22, 23, 24, 25, 26,
     27, 28, 29, 30, 31, 32, 36, 37, 38, 39, 40, 41, 42, 43, 44, 45, 46,
     47, 51, 52, 53, 54, 55, 56, 57, 58, 59, 63, 64, 65, 66, 67, 68, 75,
     76, 77, 78, 79, 80, 81, 82, 83, 87, 88, 89, 90, 91, 92],
    dtype=np.int32,
)
_IN_W = 96
_OUT_W = 66
_PAD_W = 72
_NW = 32
_PIECE = 8192


def _sc_kernel(nt: int, nb: int):
    n_jobs = nt * _OUT_W
    n_pieces = nb // _PIECE
    mesh = plsc.VectorSubcoreMesh(core_axis_name="c", subcore_axis_name="s")

    @functools.partial(
        pl.kernel,
        out_type=jax.ShapeDtypeStruct((nt * _PAD_W * nb,), jnp.float32),
        mesh=mesh,
        scratch_types=[
            pltpu.VMEM((2, _PIECE), jnp.float32),
            pltpu.VMEM((2, _PIECE), jnp.float32),
            pltpu.VMEM((_OUT_W + 14,), jnp.float32),
            pltpu.VMEM((_OUT_W + 14,), jnp.float32),
            pltpu.VMEM((_OUT_W + 14,), jnp.int32),
            pltpu.SemaphoreType.DMA,
            pltpu.SemaphoreType.DMA,
            pltpu.SemaphoreType.DMA,
            pltpu.SemaphoreType.DMA,
        ],
        compiler_params=pltpu.CompilerParams(
            needs_layout_passes=False, disable_bounds_checks=True),
    )
    def body(x_hbm, m_hbm, s_hbm, d_hbm, out_hbm, inbuf, outbuf, mbuf,
             sbuf, dbuf, sin0, sin1, sout0, sout1):
        wid = lax.axis_index("s") * mesh.num_cores + lax.axis_index("c")
        pltpu.sync_copy(m_hbm, mbuf)
        pltpu.sync_copy(s_hbm, sbuf)
        pltpu.sync_copy(d_hbm, dbuf)
        sems_in = (sin0, sin1)
        sems_out = (sout0, sout1)
        zero16 = lax.iota(jnp.int32, 16) * 0

        def src_off(job, piece):
            t = job // _OUT_W
            j = job - t * _OUT_W
            d = jnp.max(plsc.load_gather(dbuf, [zero16 + j]))
            return (t * _IN_W + d) * nb + piece * _PIECE

        def dst_off(job, piece):
            t = job // _OUT_W
            j = job - t * _OUT_W
            return (t * _PAD_W + j) * nb + piece * _PIECE

        def start_in(job, piece, slot):
            pltpu.async_copy(
                x_hbm.at[pl.ds(src_off(job, piece), _PIECE)],
                inbuf.at[slot], sems_in[slot])

        def wait_in(job, piece, slot):
            pltpu.make_async_copy(
                x_hbm.at[pl.ds(src_off(job, piece), _PIECE)],
                inbuf.at[slot], sems_in[slot]).wait()

        def start_out(job, piece, slot):
            pltpu.async_copy(
                outbuf.at[slot],
                out_hbm.at[pl.ds(dst_off(job, piece), _PIECE)],
                sems_out[slot])

        def wait_out(job, piece, slot):
            pltpu.make_async_copy(
                outbuf.at[slot],
                out_hbm.at[pl.ds(dst_off(job, piece), _PIECE)],
                sems_out[slot]).wait()

        total = n_pieces * ((n_jobs - wid + _NW - 1) // _NW)

        def glob(i):
            job = wid + (i // n_pieces) * _NW
            return job, i % n_pieces

        def do_piece(i, slot, g):
            job, piece = glob(i)
            wait_in(job, piece, slot)
            t = job // _OUT_W
            j = job - t * _OUT_W
            mv = plsc.load_gather(mbuf, [zero16 + j])
            sv = plsc.load_gather(sbuf, [zero16 + j])

            @pl.when(g >= 1)
            def _():
                jp, pp = glob(i - 2)
                wait_out(jp, pp, slot)

            def vec(v, _):
                x = inbuf[slot, pl.ds(v * 16, 16)]
                outbuf[slot, pl.ds(v * 16, 16)] = (x - mv) * sv
                return 0

            lax.fori_loop(0, _PIECE // 16, vec, 0, unroll=8)
            start_out(job, piece, slot)

        j0, p0 = glob(0)
        start_in(j0, p0, 0)

        def gstep(g, _):
            i0 = 2 * g

            @pl.when(i0 + 1 < total)
            def _():
                jn, pn = glob(i0 + 1)
                start_in(jn, pn, 1)

            do_piece(i0, 0, g)

            @pl.when(i0 + 2 < total)
            def _():
                jn, pn = glob(i0 + 2)
                start_in(jn, pn, 0)

            do_piece(i0 + 1, 1, g)
            return 0

        lax.fori_loop(0, total // 2, gstep, 0)

        jp, pp = glob(total - 2)
        wait_out(jp, pp, 0)
        jp, pp = glob(total - 1)
        wait_out(jp, pp, 1)

    return body


@jax.jit
def kernel(observed_pose, mean, std):
    b, t, w = observed_pose.shape
    xt = jnp.transpose(observed_pose, (1, 2, 0))
    xflat = xt.reshape(-1)
    m66 = mean.reshape(_OUT_W)
    rs66 = (1.0 / std).reshape(_OUT_W)
    dtab = jnp.asarray(_DIMS)
    pad = lambda a: jnp.pad(a, (0, 14))
    outflat = _sc_kernel(t, b)(xflat, pad(m66), pad(rs66), pad(dtab))
    out_t = outflat.reshape(t, _PAD_W, b)[:, :_OUT_W, :]
    return jnp.transpose(out_t, (2, 0, 1))


# speedup vs baseline: 1.6538x; 1.6538x over previous
import functools

import jax
import jax.numpy as jnp
import numpy as np
from jax import lax
from jax.experimental import pallas as pl
from jax.experimental.pallas import tpu as pltpu
from jax.experimental.pallas import tpu_sc as plsc

_DIMS = np.array(
    [6, 7, 8, 9, 10, 11, 12, 13, 14, 15, 16, 17, 21, 22, 23, 24, 25, 26,
     27, 28, 29, 30, 31, 32, 36, 37, 38, 39, 40, 41, 42, 43, 44, 45, 46,
     47, 51, 52, 53, 54, 55, 56, 57, 58, 59, 63, 64, 65, 66, 67, 68, 75,
     76, 77, 78, 79, 80, 81, 82, 83, 87, 88, 89, 90, 91, 92],
    dtype=np.int32,
)
_IN_W = 96
_OUT_W = 66
_PAD_W = 72
_NW = 32
_PIECE = 8192


def _sc_kernel(nt: int, nb: int):
    n_jobs = nt * _OUT_W
    n_pieces = nb // _PIECE
    mesh = plsc.VectorSubcoreMesh(core_axis_name="c", subcore_axis_name="s")

    @functools.partial(
        pl.kernel,
        out_type=jax.ShapeDtypeStruct((nt * _PAD_W * nb,), jnp.float32),
        mesh=mesh,
        scratch_types=[
            pltpu.VMEM((2, _PIECE), jnp.float32),
            pltpu.VMEM((2, _PIECE), jnp.float32),
            pltpu.VMEM((_OUT_W + 14,), jnp.float32),
            pltpu.VMEM((_OUT_W + 14,), jnp.float32),
            pltpu.VMEM((_OUT_W + 14,), jnp.int32),
            pltpu.SemaphoreType.DMA,
            pltpu.SemaphoreType.DMA,
            pltpu.SemaphoreType.DMA,
            pltpu.SemaphoreType.DMA,
        ],
        compiler_params=pltpu.CompilerParams(
            needs_layout_passes=False, disable_bounds_checks=True),
    )
    def body(x_hbm, m_hbm, s_hbm, d_hbm, out_hbm, inbuf, outbuf, mbuf,
             sbuf, dbuf, sin0, sin1, sout0, sout1):
        wid = lax.axis_index("s") * mesh.num_cores + lax.axis_index("c")
        pltpu.sync_copy(m_hbm, mbuf)
        pltpu.sync_copy(s_hbm, sbuf)
        pltpu.sync_copy(d_hbm, dbuf)
        sems_in = (sin0, sin1)
        sems_out = (sout0, sout1)
        zero16 = lax.iota(jnp.int32, 16) * 0

        def src_off(job, piece):
            t = job // _OUT_W
            j = job - t * _OUT_W
            d = jnp.max(plsc.load_gather(dbuf, [zero16 + j]))
            return (t * _IN_W + d) * nb + piece * _PIECE

        def dst_off(job, piece):
            t = job // _OUT_W
            j = job - t * _OUT_W
            return (t * _PAD_W + j) * nb + piece * _PIECE

        def start_in(job, piece, slot):
            pltpu.async_copy(
                x_hbm.at[pl.ds(src_off(job, piece), _PIECE)],
                inbuf.at[slot], sems_in[slot])

        def wait_in(job, piece, slot):
            pltpu.make_async_copy(
                x_hbm.at[pl.ds(src_off(job, piece), _PIECE)],
                inbuf.at[slot], sems_in[slot]).wait()

        def start_out(job, piece, slot):
            pltpu.async_copy(
                outbuf.at[slot],
                out_hbm.at[pl.ds(dst_off(job, piece), _PIECE)],
                sems_out[slot])

        def wait_out(job, piece, slot):
            pltpu.make_async_copy(
                outbuf.at[slot],
                out_hbm.at[pl.ds(dst_off(job, piece), _PIECE)],
                sems_out[slot]).wait()

        total = n_pieces * ((n_jobs - wid + _NW - 1) // _NW)

        def glob(i):
            job = wid + (i // n_pieces) * _NW
            return job, i % n_pieces

        def do_piece(i, slot, g):
            job, piece = glob(i)
            wait_in(job, piece, slot)
            t = job // _OUT_W
            j = job - t * _OUT_W
            mv = plsc.load_gather(mbuf, [zero16 + j])
            sv = plsc.load_gather(sbuf, [zero16 + j])

            @pl.when(g >= 1)
            def _():
                jp, pp = glob(i - 2)
                wait_out(jp, pp, slot)

            @plsc.parallel_loop(0, _PIECE, step=16, unroll=8)
            def _(v):
                x = inbuf[slot, pl.ds(v, 16)]
                outbuf[slot, pl.ds(v, 16)] = (x - mv) * sv

            start_out(job, piece, slot)

        j0, p0 = glob(0)
        start_in(j0, p0, 0)

        def gstep(g, _):
            i0 = 2 * g

            @pl.when(i0 + 1 < total)
            def _():
                jn, pn = glob(i0 + 1)
                start_in(jn, pn, 1)

            do_piece(i0, 0, g)

            @pl.when(i0 + 2 < total)
            def _():
                jn, pn = glob(i0 + 2)
                start_in(jn, pn, 0)

            do_piece(i0 + 1, 1, g)
            return 0

        lax.fori_loop(0, total // 2, gstep, 0)

        jp, pp = glob(total - 2)
        wait_out(jp, pp, 0)
        jp, pp = glob(total - 1)
        wait_out(jp, pp, 1)

    return body


@jax.jit
def kernel(observed_pose, mean, std):
    b, t, w = observed_pose.shape
    xt = jnp.transpose(observed_pose, (1, 2, 0))
    xflat = xt.reshape(-1)
    m66 = mean.reshape(_OUT_W)
    rs66 = (1.0 / std).reshape(_OUT_W)
    dtab = jnp.asarray(_DIMS)
    pad = lambda a: jnp.pad(a, (0, 14))
    outflat = _sc_kernel(t, b)(xflat, pad(m66), pad(rs66), pad(dtab))
    out_t = outflat.reshape(t, _PAD_W, b)[:, :_OUT_W, :]
    return jnp.transpose(out_t, (2, 0, 1))


# speedup vs baseline: 3.9430x; 2.3843x over previous
import functools

import jax
import jax.numpy as jnp
import numpy as np
from jax import lax
from jax.experimental import pallas as pl
from jax.experimental.pallas import tpu as pltpu
from jax.experimental.pallas import tpu_sc as plsc

_DIMS = np.array(
    [6, 7, 8, 9, 10, 11, 12, 13, 14, 15, 16, 17, 21, 22, 23, 24, 25, 26,
     27, 28, 29, 30, 31, 32, 36, 37, 38, 39, 40, 41, 42, 43, 44, 45, 46,
     47, 51, 52, 53, 54, 55, 56, 57, 58, 59, 63, 64, 65, 66, 67, 68, 75,
     76, 77, 78, 79, 80, 81, 82, 83, 87, 88, 89, 90, 91, 92],
    dtype=np.int32,
)
_IN_W = 96
_OUT_W = 66
_PAD_W = 72
_NW = 32
_PC = 64


def _sc_kernel(nt: int, nb: int):
    n_jobs = nt * _OUT_W
    nbt = nb // 128
    n_pieces = nbt // _PC
    in_tr = _IN_W // 8
    out_tr = _PAD_W // 8
    mesh = plsc.VectorSubcoreMesh(core_axis_name="c", subcore_axis_name="s")

    @functools.partial(
        pl.kernel,
        out_type=jax.ShapeDtypeStruct((nt * out_tr * nbt, 8, 128),
                                      jnp.float32),
        mesh=mesh,
        scratch_types=[
            pltpu.VMEM((2, _PC, 128), jnp.float32),
            pltpu.VMEM((2, _PC, 128), jnp.float32),
            pltpu.VMEM((_OUT_W + 14,), jnp.float32),
            pltpu.VMEM((_OUT_W + 14,), jnp.float32),
            pltpu.VMEM((_OUT_W + 14,), jnp.int32),
            pltpu.SemaphoreType.DMA,
            pltpu.SemaphoreType.DMA,
            pltpu.SemaphoreType.DMA,
            pltpu.SemaphoreType.DMA,
        ],
        compiler_params=pltpu.CompilerParams(
            needs_layout_passes=False, disable_bounds_checks=True),
    )
    def body(x_hbm, m_hbm, s_hbm, d_hbm, out_hbm, inbuf, outbuf, mbuf,
             sbuf, dbuf, sin0, sin1, sout0, sout1):
        wid = lax.axis_index("s") * mesh.num_cores + lax.axis_index("c")
        pltpu.sync_copy(m_hbm, mbuf)
        pltpu.sync_copy(s_hbm, sbuf)
        pltpu.sync_copy(d_hbm, dbuf)
        sems_in = (sin0, sin1)
        sems_out = (sout0, sout1)
        zero16 = lax.iota(jnp.int32, 16) * 0

        def src_ref(job, piece):
            t = job // _OUT_W
            j = job - t * _OUT_W
            d = jnp.max(plsc.load_gather(dbuf, [zero16 + j]))
            q0 = (t * in_tr + d // 8) * nbt + piece * _PC
            return x_hbm.at[pl.ds(q0, _PC), d % 8]

        def dst_ref(job, piece):
            t = job // _OUT_W
            j = job - t * _OUT_W
            q0 = (t * out_tr + j // 8) * nbt + piece * _PC
            return out_hbm.at[pl.ds(q0, _PC), j % 8]

        def start_in(job, piece, slot):
            pltpu.async_copy(src_ref(job, piece), inbuf.at[slot],
                             sems_in[slot])

        def wait_in(job, piece, slot):
            pltpu.make_async_copy(src_ref(job, piece), inbuf.at[slot],
                                  sems_in[slot]).wait()

        def start_out(job, piece, slot):
            pltpu.async_copy(outbuf.at[slot], dst_ref(job, piece),
                             sems_out[slot])

        def wait_out(job, piece, slot):
            pltpu.make_async_copy(outbuf.at[slot], dst_ref(job, piece),
                                  sems_out[slot]).wait()

        total = n_pieces * ((n_jobs - wid + _NW - 1) // _NW)

        def glob(i):
            job = wid + (i // n_pieces) * _NW
            return job, i % n_pieces

        def do_piece(i, slot, g):
            job, piece = glob(i)
            wait_in(job, piece, slot)
            t = job // _OUT_W
            j = job - t * _OUT_W
            mv = plsc.load_gather(mbuf, [zero16 + j])
            sv = plsc.load_gather(sbuf, [zero16 + j])

            @pl.when(g >= 1)
            def _():
                jp, pp = glob(i - 2)
                wait_out(jp, pp, slot)

            @plsc.parallel_loop(0, _PC, step=1, unroll=2)
            def _(r):
                for k in range(8):
                    x = inbuf[slot, r, pl.ds(16 * k, 16)]
                    outbuf[slot, r, pl.ds(16 * k, 16)] = (x - mv) * sv

            start_out(job, piece, slot)

        j0, p0 = glob(0)
        start_in(j0, p0, 0)

        def gstep(g, _):
            i0 = 2 * g

            @pl.when(i0 + 1 < total)
            def _():
                jn, pn = glob(i0 + 1)
                start_in(jn, pn, 1)

            do_piece(i0, 0, g)

            @pl.when(i0 + 2 < total)
            def _():
                jn, pn = glob(i0 + 2)
                start_in(jn, pn, 0)

            do_piece(i0 + 1, 1, g)
            return 0

        lax.fori_loop(0, total // 2, gstep, 0)

        jp, pp = glob(total - 2)
        wait_out(jp, pp, 0)
        jp, pp = glob(total - 1)
        wait_out(jp, pp, 1)

    return body


@jax.jit
def kernel(observed_pose, mean, std):
    b, t, w = observed_pose.shape
    nbt = b // 128
    x3 = (observed_pose.transpose(1, 2, 0)
          .reshape(t, w // 8, 8, nbt, 128)
          .transpose(0, 1, 3, 2, 4)
          .reshape(t * (w // 8) * nbt, 8, 128))
    m66 = mean.reshape(_OUT_W)
    rs66 = (1.0 / std).reshape(_OUT_W)
    dtab = jnp.asarray(_DIMS)
    pad = lambda a: jnp.pad(a, (0, 14))
    out3 = _sc_kernel(t, b)(x3, pad(m66), pad(rs66), pad(dtab))
    out = (out3.reshape(t, _PAD_W // 8, nbt, 8, 128)
           .transpose(0, 1, 3, 2, 4)
           .reshape(t, _PAD_W, b)[:, :_OUT_W, :]
           .transpose(2, 0, 1))
    return out


# speedup vs baseline: 4.3689x; 1.1080x over previous
import functools

import jax
import jax.numpy as jnp
import numpy as np
from jax import lax
from jax.experimental import pallas as pl
from jax.experimental.pallas import tpu as pltpu
from jax.experimental.pallas import tpu_sc as plsc

_DIMS = np.array(
    [6, 7, 8, 9, 10, 11, 12, 13, 14, 15, 16, 17, 21, 22, 23, 24, 25, 26,
     27, 28, 29, 30, 31, 32, 36, 37, 38, 39, 40, 41, 42, 43, 44, 45, 46,
     47, 51, 52, 53, 54, 55, 56, 57, 58, 59, 63, 64, 65, 66, 67, 68, 75,
     76, 77, 78, 79, 80, 81, 82, 83, 87, 88, 89, 90, 91, 92],
    dtype=np.int32,
)
_IN_W = 96
_OUT_W = 66
_PAD_W = 72
_NW = 32
_PC = 128


def _sc_kernel(nt: int, nb: int):
    n_jobs = nt * _OUT_W
    nbt = nb // 128
    n_pairs = n_jobs // 2
    in_tr = _IN_W // 8
    out_tr = _PAD_W // 8
    mesh = plsc.VectorSubcoreMesh(core_axis_name="c", subcore_axis_name="s")

    @functools.partial(
        pl.kernel,
        out_type=jax.ShapeDtypeStruct((nt * out_tr * nbt, 8, 128),
                                      jnp.float32),
        mesh=mesh,
        scratch_types=[
            pltpu.VMEM((2, _PC, 128), jnp.float32),
            pltpu.VMEM((2, _PC, 128), jnp.float32),
            pltpu.VMEM((_OUT_W + 14,), jnp.float32),
            pltpu.VMEM((_OUT_W + 14,), jnp.float32),
            pltpu.VMEM((_OUT_W + 14,), jnp.int32),
            pltpu.SemaphoreType.DMA,
            pltpu.SemaphoreType.DMA,
            pltpu.SemaphoreType.DMA,
            pltpu.SemaphoreType.DMA,
        ],
        compiler_params=pltpu.CompilerParams(
            needs_layout_passes=False, disable_bounds_checks=True),
    )
    def body(x_hbm, m_hbm, s_hbm, d_hbm, out_hbm, inbuf, outbuf, mbuf,
             sbuf, dbuf, sin0, sin1, sout0, sout1):
        wid = lax.axis_index("s") * mesh.num_cores + lax.axis_index("c")
        pltpu.sync_copy(m_hbm, mbuf)
        pltpu.sync_copy(s_hbm, sbuf)
        pltpu.sync_copy(d_hbm, dbuf)
        sems_in = (sin0, sin1)
        sems_out = (sout0, sout1)
        zero16 = lax.iota(jnp.int32, 16) * 0

        def src_ref(job, piece):
            t = job // _OUT_W
            j = job - t * _OUT_W
            d = jnp.max(plsc.load_gather(dbuf, [zero16 + j]))
            q0 = (t * in_tr + d // 8) * nbt
            return x_hbm.at[pl.ds(q0, _PC), d % 8]

        def dst_ref(job, piece):
            t = job // _OUT_W
            j = job - t * _OUT_W
            q0 = (t * out_tr + j // 8) * nbt
            return out_hbm.at[pl.ds(q0, _PC), j % 8]

        def start_in(job, piece, slot):
            pltpu.async_copy(src_ref(job, piece), inbuf.at[slot],
                             sems_in[slot])

        def wait_in(job, piece, slot):
            pltpu.make_async_copy(src_ref(job, piece), inbuf.at[slot],
                                  sems_in[slot]).wait()

        def start_out(job, piece, slot):
            pltpu.async_copy(outbuf.at[slot], dst_ref(job, piece),
                             sems_out[slot])

        def wait_out(job, piece, slot):
            pltpu.make_async_copy(outbuf.at[slot], dst_ref(job, piece),
                                  sems_out[slot]).wait()

        total = 2 * ((n_pairs - wid + _NW - 1) // _NW)

        def glob(i):
            pair = wid + (i // 2) * _NW
            job = 2 * pair + i % 2
            return job, 0

        def do_piece(i, slot, g):
            job, piece = glob(i)
            wait_in(job, piece, slot)
            t = job // _OUT_W
            j = job - t * _OUT_W
            mv = plsc.load_gather(mbuf, [zero16 + j])
            sv = plsc.load_gather(sbuf, [zero16 + j])

            @pl.when(g >= 1)
            def _():
                jp, pp = glob(i - 2)
                wait_out(jp, pp, slot)

            @plsc.parallel_loop(0, _PC, step=1, unroll=2)
            def _(r):
                for k in range(8):
                    x = inbuf[slot, r, pl.ds(16 * k, 16)]
                    outbuf[slot, r, pl.ds(16 * k, 16)] = (x - mv) * sv

            start_out(job, piece, slot)

        j0, p0 = glob(0)
        start_in(j0, p0, 0)

        def gstep(g, _):
            i0 = 2 * g

            @pl.when(i0 + 1 < total)
            def _():
                jn, pn = glob(i0 + 1)
                start_in(jn, pn, 1)

            do_piece(i0, 0, g)

            @pl.when(i0 + 2 < total)
            def _():
                jn, pn = glob(i0 + 2)
                start_in(jn, pn, 0)

            do_piece(i0 + 1, 1, g)
            return 0

        lax.fori_loop(0, total // 2, gstep, 0)

        jp, pp = glob(total - 2)
        wait_out(jp, pp, 0)
        jp, pp = glob(total - 1)
        wait_out(jp, pp, 1)

    return body


@jax.jit
def kernel(observed_pose, mean, std):
    b, t, w = observed_pose.shape
    nbt = b // 128
    x3 = (observed_pose.transpose(1, 2, 0)
          .reshape(t, w // 8, 8, nbt, 128)
          .transpose(0, 1, 3, 2, 4)
          .reshape(t * (w // 8) * nbt, 8, 128))
    m66 = mean.reshape(_OUT_W)
    rs66 = (1.0 / std).reshape(_OUT_W)
    dtab = jnp.asarray(_DIMS)
    pad = lambda a: jnp.pad(a, (0, 14))
    out3 = _sc_kernel(t, b)(x3, pad(m66), pad(rs66), pad(dtab))
    out = (out3.reshape(t, _PAD_W // 8, nbt, 8, 128)
           .transpose(0, 1, 3, 2, 4)
           .reshape(t, _PAD_W, b)[:, :_OUT_W, :]
           .transpose(2, 0, 1))
    return out
